# baseline (speedup 1.0000x reference)
import jax
import jax.numpy as jnp
import numpy as np
from jax.experimental import pallas as pl
from jax.experimental.pallas import tpu as pltpu

_B, _C1, _H, _W = 2, 96, 384, 384
_HW = _H * _W
_CS = 16
_CSE = _CS * (_CS - 1) // 2
_EPS = 1e-5
_HI, _HJ = np.triu_indices(_CS, 1)

_BH = 64
_T = _H // _BH
_BHP = 64
_TP = _H // _BHP
_TOT = _B * _TP


def _x1sum_kernel(x_ref, wf_ref, bfc_ref, buf_ref, sums_ref):
    t = pl.program_id(1)
    xb = x_ref[0].reshape(_C1, _BH * _W)
    x1 = jax.lax.dot_general(wf_ref[...], xb, (((1,), (0,)), ((), ())),
                             preferred_element_type=jnp.float32)
    x1 = x1 + bfc_ref[...]
    buf_ref[0] = x1.reshape(_C1, _BH, _W)

    @pl.when(t == 0)
    def _():
        sums_ref[...] = jnp.zeros_like(sums_ref)

    sums_ref[...] += jnp.sum(x1, axis=1).reshape(1, 1, _C1)


def _topk_kernel(sums_ref, eca_ref, idx_ref):
    m = sums_ref[:, 0, :] * (1.0 / _HW)
    z = jnp.zeros((_B, 2), jnp.float32)
    mp = jnp.concatenate([z, m, z], axis=1)
    logits = jnp.zeros_like(m)
    for k in range(5):
        logits = logits + eca_ref[k] * mp[:, k:k + _C1]
    iota_i = jax.lax.broadcasted_iota(jnp.int32, (_C1, _C1), 0)
    iota_j = jax.lax.broadcasted_iota(jnp.int32, (_C1, _C1), 1)
    iota_col = jax.lax.broadcasted_iota(jnp.int32, (_C1, 1), 0)
    for b in range(_B):
        row = logits[b:b + 1, :]
        col = jax.lax.transpose(row, (1, 0))
        mat = jnp.broadcast_to(row, (_C1, _C1))
        beats = (mat > col) | ((mat == col) & (iota_j < iota_i))
        rank = jnp.sum(beats.astype(jnp.int32), axis=1, keepdims=True)
        for k in range(_CS):
            idx_ref[b, k] = jnp.min(jnp.where(rank == k, iota_col, _C1))


def _pairs_kernel(idx_ref, buf_ref, s2_ref, b2_ref, out_ref,
                  sel2, pout2, insem, outsem):
    b = pl.program_id(0)
    t = pl.program_id(1)
    step = b * _TP + t
    slot = jax.lax.rem(step, 2)

    def in_copy(bb, tt, sl, k):
        return pltpu.make_async_copy(
            buf_ref.at[bb, idx_ref[bb, k], pl.ds(tt * _BHP, _BHP), :],
            sel2.at[sl, k], insem.at[sl])

    def out_copy(s):
        sl = jax.lax.rem(s, 2)
        sb = s // _TP
        st = jax.lax.rem(s, _TP)
        return pltpu.make_async_copy(
            pout2.at[sl],
            out_ref.at[sb, pl.ds(_C1, _CSE), pl.ds(st * _BHP, _BHP), :],
            outsem.at[sl])

    @pl.when(step == 0)
    def _():
        for k in range(_CS):
            in_copy(b, t, slot, k).start()

    @pl.when(step + 1 < _TOT)
    def _():
        ns = step + 1
        nsl = jax.lax.rem(ns, 2)
        nb = ns // _TP
        nt = jax.lax.rem(ns, _TP)
        for k in range(_CS):
            in_copy(nb, nt, nsl, k).start()

    for k in range(_CS):
        in_copy(b, t, slot, k).wait()

    @pl.when(step >= 2)
    def _():
        out_copy(step - 2).wait()

    for p in range(_CSE):
        i, j = int(_HI[p]), int(_HJ[p])
        pout2[slot, p] = (sel2[slot, i] * sel2[slot, j] * s2_ref[p]
                          + b2_ref[p])

    out_copy(step).start()

    @pl.when(step == _TOT - 1)
    def _():
        out_copy(step - 1).wait()
        out_copy(step).wait()


def kernel(x, fc_w, fc_b, bn1_gamma, bn1_beta, bn1_mean, bn1_var,
           eca_w, bn2_gamma, bn2_beta, bn2_mean, bn2_var):
    s1 = bn1_gamma * jax.lax.rsqrt(bn1_var + _EPS)
    wf = fc_w * s1[:, None]
    bf = (fc_b - bn1_mean) * s1 + bn1_beta
    s2 = bn2_gamma * jax.lax.rsqrt(bn2_var + _EPS)
    b2 = bn2_beta - bn2_mean * s2

    buf, sums = pl.pallas_call(
        _x1sum_kernel,
        grid=(_B, _T),
        in_specs=[
            pl.BlockSpec((1, _C1, _BH, _W), lambda b, t: (b, 0, t, 0)),
            pl.BlockSpec((_C1, _C1), lambda b, t: (0, 0)),
            pl.BlockSpec((_C1, 1), lambda b, t: (0, 0)),
        ],
        out_specs=[
            pl.BlockSpec((1, _C1, _BH, _W), lambda b, t: (b, 0, t, 0)),
            pl.BlockSpec((1, 1, _C1), lambda b, t: (b, 0, 0)),
        ],
        out_shape=[
            jax.ShapeDtypeStruct((_B, _C1 + _CSE, _H, _W), jnp.float32),
            jax.ShapeDtypeStruct((_B, 1, _C1), jnp.float32),
        ],
        compiler_params=pltpu.CompilerParams(
            dimension_semantics=("parallel", "arbitrary")),
        interpret=False,
    )(x, wf, bf.reshape(_C1, 1))

    idx = pl.pallas_call(
        _topk_kernel,
        in_specs=[
            pl.BlockSpec(memory_space=pltpu.VMEM),
            pl.BlockSpec(memory_space=pltpu.SMEM),
        ],
        out_specs=pl.BlockSpec(memory_space=pltpu.SMEM),
        out_shape=jax.ShapeDtypeStruct((_B, _CS), jnp.int32),
        interpret=False,
    )(sums, eca_w)

    grid_spec = pltpu.PrefetchScalarGridSpec(
        num_scalar_prefetch=1,
        grid=(_B, _TP),
        in_specs=[
            pl.BlockSpec(memory_space=pltpu.MemorySpace.HBM),
            pl.BlockSpec(memory_space=pltpu.SMEM),
            pl.BlockSpec(memory_space=pltpu.SMEM),
        ],
        out_specs=pl.BlockSpec(memory_space=pltpu.MemorySpace.HBM),
        scratch_shapes=[
            pltpu.VMEM((2, _CS, _BHP, _W), jnp.float32),
            pltpu.VMEM((2, _CSE, _BHP, _W), jnp.float32),
            pltpu.SemaphoreType.DMA((2,)),
            pltpu.SemaphoreType.DMA((2,)),
        ],
    )
    out = pl.pallas_call(
        _pairs_kernel,
        grid_spec=grid_spec,
        out_shape=jax.ShapeDtypeStruct((_B, _C1 + _CSE, _H, _W), jnp.float32),
        input_output_aliases={1: 0},
        compiler_params=pltpu.CompilerParams(
            dimension_semantics=("arbitrary", "arbitrary")),
        interpret=False,
    )(idx, buf, s2, b2)
    return out


# speedup vs baseline: 1.0091x; 1.0091x over previous
import jax
import jax.numpy as jnp
import numpy as np
from jax.experimental import pallas as pl
from jax.experimental.pallas import tpu as pltpu

_B, _C1, _H, _W = 2, 96, 384, 384
_HW = _H * _W
_CS = 16
_CSE = _CS * (_CS - 1) // 2
_EPS = 1e-5
_HI, _HJ = np.triu_indices(_CS, 1)

_BH = 48
_T = _H // _BH
_BHP = 96
_TP = _H // _BHP
_TOT = _B * _TP


def _x1sum_kernel(x_ref, wf_ref, bfc_ref, buf_ref, sums_ref):
    t = pl.program_id(1)
    xb = x_ref[0].reshape(_C1, _BH * _W)
    x1 = jax.lax.dot_general(wf_ref[...], xb, (((1,), (0,)), ((), ())),
                             preferred_element_type=jnp.float32)
    x1 = x1 + bfc_ref[...]
    buf_ref[0] = x1.reshape(_C1, _BH, _W)

    @pl.when(t == 0)
    def _():
        sums_ref[...] = jnp.zeros_like(sums_ref)

    sums_ref[...] += jnp.sum(x1, axis=1).reshape(1, 1, _C1)


def _topk_kernel(sums_ref, eca_ref, idx_ref):
    m = sums_ref[:, 0, :] * (1.0 / _HW)
    z = jnp.zeros((_B, 2), jnp.float32)
    mp = jnp.concatenate([z, m, z], axis=1)
    logits = jnp.zeros_like(m)
    for k in range(5):
        logits = logits + eca_ref[k] * mp[:, k:k + _C1]
    iota_i = jax.lax.broadcasted_iota(jnp.int32, (_C1, _C1), 0)
    iota_j = jax.lax.broadcasted_iota(jnp.int32, (_C1, _C1), 1)
    iota_col = jax.lax.broadcasted_iota(jnp.int32, (_C1, 1), 0)
    for b in range(_B):
        row = logits[b:b + 1, :]
        col = jax.lax.transpose(row, (1, 0))
        mat = jnp.broadcast_to(row, (_C1, _C1))
        beats = (mat > col) | ((mat == col) & (iota_j < iota_i))
        rank = jnp.sum(beats.astype(jnp.int32), axis=1, keepdims=True)
        for k in range(_CS):
            idx_ref[b, k] = jnp.min(jnp.where(rank == k, iota_col, _C1))


def _pairs_kernel(idx_ref, buf_ref, s2_ref, b2_ref, out_ref,
                  sel2, pout2, insem, outsem):
    b = pl.program_id(0)
    t = pl.program_id(1)
    step = b * _TP + t
    slot = jax.lax.rem(step, 2)

    def in_copy(bb, tt, sl, k):
        return pltpu.make_async_copy(
            buf_ref.at[bb, idx_ref[bb, k], pl.ds(tt * _BHP, _BHP), :],
            sel2.at[sl, k], insem.at[sl])

    def out_copy(s):
        sl = jax.lax.rem(s, 2)
        sb = s // _TP
        st = jax.lax.rem(s, _TP)
        return pltpu.make_async_copy(
            pout2.at[sl],
            out_ref.at[sb, pl.ds(_C1, _CSE), pl.ds(st * _BHP, _BHP), :],
            outsem.at[sl])

    @pl.when(step == 0)
    def _():
        for k in range(_CS):
            in_copy(b, t, slot, k).start()

    @pl.when(step + 1 < _TOT)
    def _():
        ns = step + 1
        nsl = jax.lax.rem(ns, 2)
        nb = ns // _TP
        nt = jax.lax.rem(ns, _TP)
        for k in range(_CS):
            in_copy(nb, nt, nsl, k).start()

    for k in range(_CS):
        in_copy(b, t, slot, k).wait()

    @pl.when(step >= 2)
    def _():
        out_copy(step - 2).wait()

    for p in range(_CSE):
        i, j = int(_HI[p]), int(_HJ[p])
        pout2[slot, p] = (sel2[slot, i] * sel2[slot, j] * s2_ref[p]
                          + b2_ref[p])

    out_copy(step).start()

    @pl.when(step == _TOT - 1)
    def _():
        out_copy(step - 1).wait()
        out_copy(step).wait()


def kernel(x, fc_w, fc_b, bn1_gamma, bn1_beta, bn1_mean, bn1_var,
           eca_w, bn2_gamma, bn2_beta, bn2_mean, bn2_var):
    s1 = bn1_gamma * jax.lax.rsqrt(bn1_var + _EPS)
    wf = fc_w * s1[:, None]
    bf = (fc_b - bn1_mean) * s1 + bn1_beta
    s2 = bn2_gamma * jax.lax.rsqrt(bn2_var + _EPS)
    b2 = bn2_beta - bn2_mean * s2

    buf, sums = pl.pallas_call(
        _x1sum_kernel,
        grid=(_B, _T),
        in_specs=[
            pl.BlockSpec((1, _C1, _BH, _W), lambda b, t: (b, 0, t, 0)),
            pl.BlockSpec((_C1, _C1), lambda b, t: (0, 0)),
            pl.BlockSpec((_C1, 1), lambda b, t: (0, 0)),
        ],
        out_specs=[
            pl.BlockSpec((1, _C1, _BH, _W), lambda b, t: (b, 0, t, 0)),
            pl.BlockSpec((1, 1, _C1), lambda b, t: (b, 0, 0)),
        ],
        out_shape=[
            jax.ShapeDtypeStruct((_B, _C1 + _CSE, _H, _W), jnp.float32),
            jax.ShapeDtypeStruct((_B, 1, _C1), jnp.float32),
        ],
        compiler_params=pltpu.CompilerParams(
            dimension_semantics=("parallel", "arbitrary")),
        interpret=False,
    )(x, wf, bf.reshape(_C1, 1))

    idx = pl.pallas_call(
        _topk_kernel,
        in_specs=[
            pl.BlockSpec(memory_space=pltpu.VMEM),
            pl.BlockSpec(memory_space=pltpu.SMEM),
        ],
        out_specs=pl.BlockSpec(memory_space=pltpu.SMEM),
        out_shape=jax.ShapeDtypeStruct((_B, _CS), jnp.int32),
        interpret=False,
    )(sums, eca_w)

    grid_spec = pltpu.PrefetchScalarGridSpec(
        num_scalar_prefetch=1,
        grid=(_B, _TP),
        in_specs=[
            pl.BlockSpec(memory_space=pltpu.MemorySpace.HBM),
            pl.BlockSpec(memory_space=pltpu.SMEM),
            pl.BlockSpec(memory_space=pltpu.SMEM),
        ],
        out_specs=pl.BlockSpec(memory_space=pltpu.MemorySpace.HBM),
        scratch_shapes=[
            pltpu.VMEM((2, _CS, _BHP, _W), jnp.float32),
            pltpu.VMEM((2, _CSE, _BHP, _W), jnp.float32),
            pltpu.SemaphoreType.DMA((2,)),
            pltpu.SemaphoreType.DMA((2,)),
        ],
    )
    out = pl.pallas_call(
        _pairs_kernel,
        grid_spec=grid_spec,
        out_shape=jax.ShapeDtypeStruct((_B, _C1 + _CSE, _H, _W), jnp.float32),
        input_output_aliases={1: 0},
        compiler_params=pltpu.CompilerParams(
            dimension_semantics=("arbitrary", "arbitrary")),
        interpret=False,
    )(idx, buf, s2, b2)
    return out
